# Initial kernel scaffold; baseline (speedup 1.0000x reference)
import jax
import jax.numpy as jnp
from jax.experimental import pallas as pl


def kernel(H, Wc, bc, W1, b1, W2, b2, alpha):
    raise NotImplementedError("write your pallas kernel here")


# speedup vs baseline: 6.7274x; 6.7274x over previous
import functools

import jax
import jax.numpy as jnp
from jax.experimental import pallas as pl


def _router_kernel(h_ref, wc_ref, bc_ref, w1_ref, b1_ref, w2_ref, b2_ref,
                   alpha_ref, w_out_ref, l_out_ref, *, block_rows, k):
    i = pl.program_id(0)
    h_all = h_ref[...]
    norm = jnp.sqrt(jnp.sum(h_all * h_all, axis=1, keepdims=True))
    hn_all = h_all / (norm + 1e-8)
    h_blk = h_ref[pl.ds(i * block_rows, block_rows), :]
    nrm_b = jnp.sqrt(jnp.sum(h_blk * h_blk, axis=1, keepdims=True))
    hn_blk = h_blk / (nrm_b + 1e-8)

    sim = jax.lax.dot_general(
        hn_blk, hn_all,
        dimension_numbers=(((1,), (1,)), ((), ())),
        preferred_element_type=jnp.float32)
    dist = 1.0 - sim

    kp1 = float(k + 1)
    taken = jnp.zeros((block_rows, 1), jnp.float32)
    ms, ovs = [], []
    max_last = jnp.zeros((block_rows, 1), jnp.float32)
    for _ in range(k + 1):
        active = taken < kp1
        m = jnp.min(dist, axis=1, keepdims=True)
        m = jnp.where(active, m, 0.0)
        eq = dist == m
        cnt = jnp.sum(eq.astype(jnp.float32), axis=1, keepdims=True)
        dist = jnp.where(eq & active, jnp.inf, dist)
        lo = jnp.maximum(taken + 1.0, 2.0)
        hi = jnp.minimum(taken + cnt, kp1)
        ov = jnp.where(active, jnp.clip(hi - lo + 1.0, 0.0, float(k)), 0.0)
        ms.append(m)
        ovs.append(ov)
        max_last = jnp.where(active & (taken + cnt >= kp1), m, max_last)
        taken = taken + jnp.where(active, cnt, 0.0)

    ksum = jnp.zeros((block_rows, 1), jnp.float32)
    for m, ov in zip(ms, ovs):
        ksum += ov * m
    mean = ksum / float(k)
    var = jnp.zeros((block_rows, 1), jnp.float32)
    for m, ov in zip(ms, ovs):
        dmu = m - mean
        var += ov * dmu * dmu
    std = jnp.sqrt(var / float(k - 1))
    outlier = max_last / (mean + 1e-8)

    w1 = w1_ref[...]
    hid = (std * w1[:, 0][None, :] + mean * w1[:, 1][None, :]
           + outlier * w1[:, 2][None, :] + b1_ref[...])
    hid = jnp.maximum(hid, 0.0)
    topo = jax.lax.dot_general(
        hid, w2_ref[...],
        dimension_numbers=(((1,), (1,)), ((), ())),
        preferred_element_type=jnp.float32) + b2_ref[...]

    content = jax.lax.dot_general(
        h_blk, wc_ref[...],
        dimension_numbers=(((1,), (1,)), ((), ())),
        preferred_element_type=jnp.float32) + bc_ref[...]

    mix = jax.nn.sigmoid(alpha_ref[0, 0])
    logits = mix * content + (1.0 - mix) * topo
    zmax = jnp.max(logits, axis=1, keepdims=True)
    ez = jnp.exp(logits - zmax)
    weights = ez / jnp.sum(ez, axis=1, keepdims=True)

    w_out_ref[...] = weights
    l_out_ref[...] = logits


def kernel(H, Wc, bc, W1, b1, W2, b2, alpha):
    b, s, d = H.shape
    n = b * s
    ng = Wc.shape[0]
    nh = W1.shape[0]
    k = min(8, n - 1)

    block_rows = 256
    while n % block_rows != 0:
        block_rows //= 2
    grid = n // block_rows

    hf = H.reshape(n, d)
    body = functools.partial(_router_kernel, block_rows=block_rows, k=k)
    weights, logits = pl.pallas_call(
        body,
        grid=(grid,),
        in_specs=[
            pl.BlockSpec((n, d), lambda i: (0, 0)),
            pl.BlockSpec((ng, d), lambda i: (0, 0)),
            pl.BlockSpec((1, ng), lambda i: (0, 0)),
            pl.BlockSpec((nh, 3), lambda i: (0, 0)),
            pl.BlockSpec((1, nh), lambda i: (0, 0)),
            pl.BlockSpec((ng, nh), lambda i: (0, 0)),
            pl.BlockSpec((1, ng), lambda i: (0, 0)),
            pl.BlockSpec((1, 1), lambda i: (0, 0)),
        ],
        out_specs=[
            pl.BlockSpec((block_rows, ng), lambda i: (i, 0)),
            pl.BlockSpec((block_rows, ng), lambda i: (i, 0)),
        ],
        out_shape=[
            jax.ShapeDtypeStruct((n, ng), jnp.float32),
            jax.ShapeDtypeStruct((n, ng), jnp.float32),
        ],
    )(hf, Wc, bc.reshape(1, ng), W1, b1.reshape(1, nh), W2,
      b2.reshape(1, ng), alpha.reshape(1, 1))

    return weights.reshape(b, s, ng), logits.reshape(b, s, ng)


# speedup vs baseline: 7.1159x; 1.0578x over previous
import functools

import jax
import jax.numpy as jnp
from jax.experimental import pallas as pl


def _normalize_kernel(h_ref, hn_ref):
    h = h_ref[...]
    norm = jnp.sqrt(jnp.sum(h * h, axis=1, keepdims=True))
    hn_ref[...] = h / (norm + 1e-8)


def _router_kernel(h_ref, hn_full_ref, hn_blk_ref, wc_ref, bc_ref, w1_ref,
                   b1_ref, w2_ref, b2_ref, alpha_ref, w_out_ref, l_out_ref,
                   *, block_rows, k):
    h_blk = h_ref[...]
    hn_blk = hn_blk_ref[...]

    sim = jax.lax.dot_general(
        hn_blk, hn_full_ref[...],
        dimension_numbers=(((1,), (1,)), ((), ())),
        preferred_element_type=jnp.float32)
    dist = 1.0 - sim

    kp1 = float(k + 1)
    taken = jnp.zeros((block_rows, 1), jnp.float32)
    ms, ovs = [], []
    max_last = jnp.zeros((block_rows, 1), jnp.float32)
    for _ in range(k + 1):
        active = taken < kp1
        m = jnp.min(dist, axis=1, keepdims=True)
        m = jnp.where(active, m, 0.0)
        eq = dist == m
        cnt = jnp.sum(eq.astype(jnp.float32), axis=1, keepdims=True)
        dist = jnp.where(eq & active, jnp.inf, dist)
        lo = jnp.maximum(taken + 1.0, 2.0)
        hi = jnp.minimum(taken + cnt, kp1)
        ov = jnp.where(active, jnp.clip(hi - lo + 1.0, 0.0, float(k)), 0.0)
        ms.append(m)
        ovs.append(ov)
        max_last = jnp.where(active & (taken + cnt >= kp1), m, max_last)
        taken = taken + jnp.where(active, cnt, 0.0)

    ksum = jnp.zeros((block_rows, 1), jnp.float32)
    for m, ov in zip(ms, ovs):
        ksum += ov * m
    mean = ksum / float(k)
    var = jnp.zeros((block_rows, 1), jnp.float32)
    for m, ov in zip(ms, ovs):
        dmu = m - mean
        var += ov * dmu * dmu
    std = jnp.sqrt(var / float(k - 1))
    outlier = max_last / (mean + 1e-8)

    w1 = w1_ref[...]
    hid = (std * w1[:, 0][None, :] + mean * w1[:, 1][None, :]
           + outlier * w1[:, 2][None, :] + b1_ref[...])
    hid = jnp.maximum(hid, 0.0)
    topo = jax.lax.dot_general(
        hid, w2_ref[...],
        dimension_numbers=(((1,), (1,)), ((), ())),
        preferred_element_type=jnp.float32) + b2_ref[...]

    content = jax.lax.dot_general(
        h_blk, wc_ref[...],
        dimension_numbers=(((1,), (1,)), ((), ())),
        preferred_element_type=jnp.float32) + bc_ref[...]

    mix = jax.nn.sigmoid(alpha_ref[0, 0])
    logits = mix * content + (1.0 - mix) * topo
    zmax = jnp.max(logits, axis=1, keepdims=True)
    ez = jnp.exp(logits - zmax)
    weights = ez / jnp.sum(ez, axis=1, keepdims=True)

    w_out_ref[...] = weights
    l_out_ref[...] = logits


def kernel(H, Wc, bc, W1, b1, W2, b2, alpha):
    b, s, d = H.shape
    n = b * s
    ng = Wc.shape[0]
    nh = W1.shape[0]
    k = min(8, n - 1)

    block_rows = 256
    while n % block_rows != 0:
        block_rows //= 2
    grid = n // block_rows

    hf = H.reshape(n, d)

    norm_rows = min(n, 1024)
    hn = pl.pallas_call(
        _normalize_kernel,
        grid=(n // norm_rows,),
        in_specs=[pl.BlockSpec((norm_rows, d), lambda i: (i, 0))],
        out_specs=pl.BlockSpec((norm_rows, d), lambda i: (i, 0)),
        out_shape=jax.ShapeDtypeStruct((n, d), jnp.float32),
    )(hf)

    body = functools.partial(_router_kernel, block_rows=block_rows, k=k)
    weights, logits = pl.pallas_call(
        body,
        grid=(grid,),
        in_specs=[
            pl.BlockSpec((block_rows, d), lambda i: (i, 0)),
            pl.BlockSpec((n, d), lambda i: (0, 0)),
            pl.BlockSpec((block_rows, d), lambda i: (i, 0)),
            pl.BlockSpec((ng, d), lambda i: (0, 0)),
            pl.BlockSpec((1, ng), lambda i: (0, 0)),
            pl.BlockSpec((nh, 3), lambda i: (0, 0)),
            pl.BlockSpec((1, nh), lambda i: (0, 0)),
            pl.BlockSpec((ng, nh), lambda i: (0, 0)),
            pl.BlockSpec((1, ng), lambda i: (0, 0)),
            pl.BlockSpec((1, 1), lambda i: (0, 0)),
        ],
        out_specs=[
            pl.BlockSpec((block_rows, ng), lambda i: (i, 0)),
            pl.BlockSpec((block_rows, ng), lambda i: (i, 0)),
        ],
        out_shape=[
            jax.ShapeDtypeStruct((n, ng), jnp.float32),
            jax.ShapeDtypeStruct((n, ng), jnp.float32),
        ],
    )(hf, hn, hn, Wc, bc.reshape(1, ng), W1, b1.reshape(1, nh), W2,
      b2.reshape(1, ng), alpha.reshape(1, 1))

    return weights.reshape(b, s, ng), logits.reshape(b, s, ng)


# speedup vs baseline: 9.2273x; 1.2967x over previous
import functools

import jax
import jax.numpy as jnp
from jax.experimental import pallas as pl
from jax.experimental.pallas import tpu as pltpu


def _normalize_kernel(h_ref, hn_ref):
    h = h_ref[...]
    norm = jnp.sqrt(jnp.sum(h * h, axis=1, keepdims=True))
    hn_ref[...] = h / (norm + 1e-8)


def _router_kernel(h_ref, hn_full_ref, hn_blk_ref, wc_ref, bc_ref, w1_ref,
                   b1_ref, w2_ref, b2_ref, alpha_ref, w_out_ref, l_out_ref,
                   dscr_ref, cscr_ref, *, block_rows, k):
    n = hn_full_ref.shape[0]
    k2 = k + 1
    h_blk = h_ref[...]
    hn_blk = hn_blk_ref[...]

    sim = jax.lax.dot_general(
        hn_blk, hn_full_ref[...],
        dimension_numbers=(((1,), (1,)), ((), ())),
        preferred_element_type=jnp.float32)
    dscr_ref[...] = 1.0 - sim

    n_chunks = n // 128
    unroll = 4
    while n_chunks % unroll:
        unroll //= 2
    for rg in range(block_rows // 8):
        r0 = rg * 8

        def body(ci, regs, r0=r0):
            regs = list(regs)
            for u in range(unroll):
                x = dscr_ref[pl.ds(r0, 8), pl.ds((ci * unroll + u) * 128, 128)]
                for j in range(k2):
                    lo = jnp.minimum(regs[j], x)
                    x = jnp.maximum(regs[j], x)
                    regs[j] = lo
            return tuple(regs)

        regs = jax.lax.fori_loop(
            0, n_chunks // unroll, body,
            tuple(jnp.full((8, 128), jnp.inf, jnp.float32) for _ in range(k2)))
        for j in range(k2):
            cscr_ref[pl.ds(r0, 8), pl.ds(j * 128, 128)] = regs[j]

    dist = cscr_ref[...]

    kp1 = float(k + 1)
    taken = jnp.zeros((block_rows, 1), jnp.float32)
    ms, ovs = [], []
    max_last = jnp.zeros((block_rows, 1), jnp.float32)
    for _ in range(k + 1):
        active = taken < kp1
        m = jnp.min(dist, axis=1, keepdims=True)
        m = jnp.where(active, m, 0.0)
        eq = dist == m
        cnt = jnp.sum(eq.astype(jnp.float32), axis=1, keepdims=True)
        dist = jnp.where(eq & active, jnp.inf, dist)
        lo = jnp.maximum(taken + 1.0, 2.0)
        hi = jnp.minimum(taken + cnt, kp1)
        ov = jnp.where(active, jnp.clip(hi - lo + 1.0, 0.0, float(k)), 0.0)
        ms.append(m)
        ovs.append(ov)
        max_last = jnp.where(active & (taken + cnt >= kp1), m, max_last)
        taken = taken + jnp.where(active, cnt, 0.0)

    ksum = jnp.zeros((block_rows, 1), jnp.float32)
    for m, ov in zip(ms, ovs):
        ksum += ov * m
    mean = ksum / float(k)
    var = jnp.zeros((block_rows, 1), jnp.float32)
    for m, ov in zip(ms, ovs):
        dmu = m - mean
        var += ov * dmu * dmu
    std = jnp.sqrt(var / float(k - 1))
    outlier = max_last / (mean + 1e-8)

    w1 = w1_ref[...]
    hid = (std * w1[:, 0][None, :] + mean * w1[:, 1][None, :]
           + outlier * w1[:, 2][None, :] + b1_ref[...])
    hid = jnp.maximum(hid, 0.0)
    topo = jax.lax.dot_general(
        hid, w2_ref[...],
        dimension_numbers=(((1,), (1,)), ((), ())),
        preferred_element_type=jnp.float32) + b2_ref[...]

    content = jax.lax.dot_general(
        h_blk, wc_ref[...],
        dimension_numbers=(((1,), (1,)), ((), ())),
        preferred_element_type=jnp.float32) + bc_ref[...]

    mix = jax.nn.sigmoid(alpha_ref[0, 0])
    logits = mix * content + (1.0 - mix) * topo
    zmax = jnp.max(logits, axis=1, keepdims=True)
    ez = jnp.exp(logits - zmax)
    weights = ez / jnp.sum(ez, axis=1, keepdims=True)

    w_out_ref[...] = weights
    l_out_ref[...] = logits


def kernel(H, Wc, bc, W1, b1, W2, b2, alpha):
    b, s, d = H.shape
    n = b * s
    ng = Wc.shape[0]
    nh = W1.shape[0]
    k = min(8, n - 1)

    block_rows = 256
    while n % block_rows != 0:
        block_rows //= 2
    grid = n // block_rows

    hf = H.reshape(n, d)

    norm_rows = min(n, 1024)
    hn = pl.pallas_call(
        _normalize_kernel,
        grid=(n // norm_rows,),
        in_specs=[pl.BlockSpec((norm_rows, d), lambda i: (i, 0))],
        out_specs=pl.BlockSpec((norm_rows, d), lambda i: (i, 0)),
        out_shape=jax.ShapeDtypeStruct((n, d), jnp.float32),
    )(hf)

    body = functools.partial(_router_kernel, block_rows=block_rows, k=k)
    weights, logits = pl.pallas_call(
        body,
        grid=(grid,),
        in_specs=[
            pl.BlockSpec((block_rows, d), lambda i: (i, 0)),
            pl.BlockSpec((n, d), lambda i: (0, 0)),
            pl.BlockSpec((block_rows, d), lambda i: (i, 0)),
            pl.BlockSpec((ng, d), lambda i: (0, 0)),
            pl.BlockSpec((1, ng), lambda i: (0, 0)),
            pl.BlockSpec((nh, 3), lambda i: (0, 0)),
            pl.BlockSpec((1, nh), lambda i: (0, 0)),
            pl.BlockSpec((ng, nh), lambda i: (0, 0)),
            pl.BlockSpec((1, ng), lambda i: (0, 0)),
            pl.BlockSpec((1, 1), lambda i: (0, 0)),
        ],
        out_specs=[
            pl.BlockSpec((block_rows, ng), lambda i: (i, 0)),
            pl.BlockSpec((block_rows, ng), lambda i: (i, 0)),
        ],
        out_shape=[
            jax.ShapeDtypeStruct((n, ng), jnp.float32),
            jax.ShapeDtypeStruct((n, ng), jnp.float32),
        ],
        scratch_shapes=[
            pltpu.VMEM((block_rows, n), jnp.float32),
            pltpu.VMEM((block_rows, (k + 1) * 128), jnp.float32),
        ],
    )(hf, hn, hn, Wc, bc.reshape(1, ng), W1, b1.reshape(1, nh), W2,
      b2.reshape(1, ng), alpha.reshape(1, 1))

    return weights.reshape(b, s, ng), logits.reshape(b, s, ng)
